# Initial kernel scaffold; baseline (speedup 1.0000x reference)
import jax
import jax.numpy as jnp
from jax.experimental import pallas as pl


def kernel(x, edge_index, edge_weight, noise, W1, W2, W3):
    raise NotImplementedError("write your pallas kernel here")


# speedup vs baseline: 6.0572x; 6.0572x over previous
import functools

import jax
import jax.numpy as jnp
from jax import lax
from jax.experimental import pallas as pl
from jax.experimental.pallas import tpu as pltpu
from jax.experimental.pallas import tpu_sc as plsc

N_NODES = 10000
N_EDGES = 160000
N_FEATS = 128
N_HID1 = 32
N_HID2 = 16

NC = 2
NS = 16
NW = NC * NS
CHUNK = 128
N_CHUNKS = N_EDGES // CHUNK
FULL_ROUNDS = N_CHUNKS // NW
REM_CHUNKS = N_CHUNKS - FULL_ROUNDS * NW
ROWS_A = 624
ROWS_B = N_NODES - 15 * ROWS_A


_SPMM_SCRATCH = [
    pltpu.VMEM_SHARED((N_NODES, N_HID1), jnp.float32),
    pltpu.VMEM((CHUNK,), jnp.int32),
    pltpu.VMEM((CHUNK,), jnp.int32),
    pltpu.VMEM((CHUNK,), jnp.float32),
    pltpu.VMEM((CHUNK, N_HID1), jnp.float32),
    pltpu.VMEM((ROWS_B, N_HID1), jnp.float32),
]


def _spmm_body(m_hbm, src_hbm, dst_hbm, w_hbm, out_hbm,
               acc_shared, src_v, dst_v, w_v, rows_v, buf_v):
    cid = lax.axis_index("c")
    sid = lax.axis_index("s")
    wid = sid * NC + cid

    zero16 = jnp.zeros((16,), jnp.float32)

    def zero_row(r, carry):
        buf_v[r, 0:16] = zero16
        buf_v[r, 16:32] = zero16
        return carry

    lax.fori_loop(0, ROWS_B, zero_row, 0)

    @pl.when(sid < 15)
    def _():
        pltpu.sync_copy(buf_v.at[pl.ds(0, ROWS_A), :],
                        acc_shared.at[pl.ds(sid * ROWS_A, ROWS_A), :])

    @pl.when(sid == 15)
    def _():
        pltpu.sync_copy(buf_v, acc_shared.at[pl.ds(15 * ROWS_A, ROWS_B), :])

    plsc.subcore_barrier()

    def process_chunk(chunk_id):
        base = chunk_id * CHUNK
        pltpu.sync_copy(src_hbm.at[pl.ds(base, CHUNK)], src_v)
        pltpu.sync_copy(dst_hbm.at[pl.ds(base, CHUNK)], dst_v)
        pltpu.sync_copy(w_hbm.at[pl.ds(base, CHUNK)], w_v)
        pltpu.sync_copy(m_hbm.at[src_v], rows_v)

        def scale_group(j, carry):
            w16 = w_v[pl.ds(j * 16, 16)]
            for i in range(16):
                e = j * 16 + i
                w = w16[i]
                rows_v[e, 0:16] = rows_v[e, 0:16] * w
                rows_v[e, 16:32] = rows_v[e, 16:32] * w
            return carry

        lax.fori_loop(0, CHUNK // 16, scale_group, 0)
        pltpu.sync_copy(rows_v, acc_shared.at[dst_v], add=True)

    def round_body(k, carry):
        process_chunk(k * NW + wid)
        return carry

    lax.fori_loop(0, FULL_ROUNDS, round_body, 0)

    @pl.when(wid < REM_CHUNKS)
    def _():
        process_chunk(FULL_ROUNDS * NW + wid)

    plsc.subcore_barrier()

    @pl.when(sid < 15)
    def _():
        pltpu.sync_copy(acc_shared.at[pl.ds(sid * ROWS_A, ROWS_A), :],
                        buf_v.at[pl.ds(0, ROWS_A), :])
        pltpu.sync_copy(buf_v.at[pl.ds(0, ROWS_A), :],
                        out_hbm.at[cid, pl.ds(sid * ROWS_A, ROWS_A), :])

    @pl.when(sid == 15)
    def _():
        pltpu.sync_copy(acc_shared.at[pl.ds(15 * ROWS_A, ROWS_B), :], buf_v)
        pltpu.sync_copy(buf_v, out_hbm.at[cid, pl.ds(15 * ROWS_A, ROWS_B), :])


@functools.lru_cache(maxsize=None)
def _get_spmm_sc():
    return pl.kernel(
        _spmm_body,
        out_type=jax.ShapeDtypeStruct((NC, N_NODES, N_HID1), jnp.float32),
        mesh=plsc.VectorSubcoreMesh(
            core_axis_name="c", subcore_axis_name="s",
            num_cores=NC, num_subcores=NS),
        compiler_params=pltpu.CompilerParams(use_tc_tiling_on_sc=False),
        scratch_types=_SPMM_SCRATCH,
    )


def _xw_body(x_ref, w_ref, o_ref):
    o_ref[...] = jnp.dot(x_ref[...], w_ref[...],
                         preferred_element_type=jnp.float32)


def _u_body(p_ref, w23_ref, o_ref):
    h = jnp.maximum(p_ref[0] + p_ref[1], 0.0)
    o_ref[...] = jnp.dot(h, w23_ref[...], preferred_element_type=jnp.float32)


def _z_body(q_ref, noise_ref, o_ref):
    m = q_ref[0] + q_ref[1]
    o_ref[...] = m[:, :N_HID2] + jnp.exp(m[:, N_HID2:]) * noise_ref[...]


BM = 200


def _decode_body(zi_ref, zj_ref, o_ref):
    s = lax.dot_general(zi_ref[...], zj_ref[...], (((1,), (1,)), ((), ())),
                        preferred_element_type=jnp.float32)
    o_ref[...] = jax.nn.sigmoid(s)


def kernel(x, edge_index, edge_weight, noise, W1, W2, W3):
    src = edge_index[0].astype(jnp.int32)
    dst = edge_index[1].astype(jnp.int32)
    w = edge_weight.astype(jnp.float32)
    w23 = jnp.concatenate([W2, W3], axis=1)

    spmm = _get_spmm_sc()

    xw = pl.pallas_call(
        _xw_body,
        out_shape=jax.ShapeDtypeStruct((N_NODES, N_HID1), jnp.float32),
    )(x, W1)

    p = spmm(xw, src, dst, w)

    u = pl.pallas_call(
        _u_body,
        out_shape=jax.ShapeDtypeStruct((N_NODES, N_HID1), jnp.float32),
    )(p, w23)

    q = spmm(u, src, dst, w)

    z = pl.pallas_call(
        _z_body,
        out_shape=jax.ShapeDtypeStruct((N_NODES, N_HID2), jnp.float32),
    )(q, noise)

    a_pred = pl.pallas_call(
        _decode_body,
        grid=(N_NODES // BM,),
        in_specs=[
            pl.BlockSpec((BM, N_HID2), lambda i: (i, 0)),
            pl.BlockSpec((N_NODES, N_HID2), lambda i: (0, 0)),
        ],
        out_specs=pl.BlockSpec((BM, N_NODES), lambda i: (i, 0)),
        out_shape=jax.ShapeDtypeStruct((N_NODES, N_NODES), jnp.float32),
    )(z, z)

    return a_pred


# speedup vs baseline: 7.4164x; 1.2244x over previous
import functools

import jax
import jax.numpy as jnp
from jax import lax
from jax.experimental import pallas as pl
from jax.experimental.pallas import tpu as pltpu
from jax.experimental.pallas import tpu_sc as plsc

N_NODES = 10000
N_EDGES = 160000
N_FEATS = 128
N_HID1 = 32
N_HID2 = 16

NC = 2
NS = 16
NW = NC * NS
CHUNK = 128
CHUNKS_PER_TILE = 40
EROWS = NW * CHUNKS_PER_TILE
E_PAD = EROWS * CHUNK
NBUF = 4
ROWS_A = 624
ROWS_B = N_NODES - 15 * ROWS_A


_SPMM_SCRATCH = (
    [pltpu.VMEM_SHARED((N_NODES, N_HID1), jnp.float32)]
    + [pltpu.VMEM((CHUNKS_PER_TILE, CHUNK), jnp.int32)] * 2
    + [pltpu.VMEM((CHUNKS_PER_TILE, CHUNK), jnp.float32)]
    + [pltpu.VMEM((CHUNK, N_HID1), jnp.float32)] * NBUF
    + [pltpu.SemaphoreType.DMA] * (2 * NBUF)
    + [pltpu.VMEM((ROWS_B, N_HID1), jnp.float32)]
)


def _spmm_body(m_hbm, src_hbm, dst_hbm, w_hbm, out_hbm,
               acc_shared, src_all, dst_all, w_all,
               rows0, rows1, rows2, rows3,
               g0, g1, g2, g3, s0, s1, s2, s3, buf_v):
    rows = [rows0, rows1, rows2, rows3]
    gsem = [g0, g1, g2, g3]
    ssem = [s0, s1, s2, s3]
    cid = lax.axis_index("c")
    sid = lax.axis_index("s")
    wid = sid * NC + cid
    row0 = wid * CHUNKS_PER_TILE

    pltpu.sync_copy(src_hbm.at[pl.ds(row0, CHUNKS_PER_TILE), :], src_all)
    pltpu.sync_copy(dst_hbm.at[pl.ds(row0, CHUNKS_PER_TILE), :], dst_all)
    pltpu.sync_copy(w_hbm.at[pl.ds(row0, CHUNKS_PER_TILE), :], w_all)

    zero16 = jnp.zeros((16,), jnp.float32)

    def zero_row(r, carry):
        buf_v[r, 0:16] = zero16
        buf_v[r, 16:32] = zero16
        return carry

    lax.fori_loop(0, ROWS_B, zero_row, 0)

    @pl.when(sid < 15)
    def _():
        pltpu.sync_copy(buf_v.at[pl.ds(0, ROWS_A), :],
                        acc_shared.at[pl.ds(sid * ROWS_A, ROWS_A), :])

    @pl.when(sid == 15)
    def _():
        pltpu.sync_copy(buf_v, acc_shared.at[pl.ds(15 * ROWS_A, ROWS_B), :])

    plsc.subcore_barrier()

    def gather_start(c, b):
        pltpu.async_copy(m_hbm.at[src_all.at[c]], rows[b], gsem[b])

    def wait_chunk(sem, b):
        pltpu.make_async_copy(m_hbm.at[pl.ds(0, CHUNK), :], rows[b], sem).wait()

    def scale_rows(c, b):
        def scale_group(j, carry):
            w16 = w_all[c, pl.ds(j * 16, 16)]
            for i in range(16):
                e = j * 16 + i
                w = w16[i]
                rows[b][e, 0:16] = rows[b][e, 0:16] * w
                rows[b][e, 16:32] = rows[b][e, 16:32] * w
            return carry

        lax.fori_loop(0, CHUNK // 16, scale_group, 0)

    for b in range(NBUF - 1):
        gather_start(b, b)

    def outer(kk, carry):
        for b in range(NBUF):
            c = kk * NBUF + b
            bp = (b + NBUF - 1) % NBUF

            @pl.when(c + NBUF - 1 < CHUNKS_PER_TILE)
            def _():
                @pl.when(c >= 1)
                def _():
                    wait_chunk(ssem[bp], bp)

                gather_start(c + NBUF - 1, bp)

            wait_chunk(gsem[b], b)
            scale_rows(c, b)
            pltpu.async_copy(rows[b], acc_shared.at[dst_all.at[c]],
                             ssem[b], add=True)
        return carry

    lax.fori_loop(0, CHUNKS_PER_TILE // NBUF, outer, 0)

    for b in range(NBUF):
        wait_chunk(ssem[b], b)

    plsc.subcore_barrier()

    @pl.when(sid < 15)
    def _():
        pltpu.sync_copy(acc_shared.at[pl.ds(sid * ROWS_A, ROWS_A), :],
                        buf_v.at[pl.ds(0, ROWS_A), :])
        pltpu.sync_copy(buf_v.at[pl.ds(0, ROWS_A), :],
                        out_hbm.at[cid, pl.ds(sid * ROWS_A, ROWS_A), :])

    @pl.when(sid == 15)
    def _():
        pltpu.sync_copy(acc_shared.at[pl.ds(15 * ROWS_A, ROWS_B), :], buf_v)
        pltpu.sync_copy(buf_v, out_hbm.at[cid, pl.ds(15 * ROWS_A, ROWS_B), :])


@functools.lru_cache(maxsize=None)
def _get_spmm_sc():
    return pl.kernel(
        _spmm_body,
        out_type=jax.ShapeDtypeStruct((NC, N_NODES, N_HID1), jnp.float32),
        mesh=plsc.VectorSubcoreMesh(
            core_axis_name="c", subcore_axis_name="s",
            num_cores=NC, num_subcores=NS),
        compiler_params=pltpu.CompilerParams(use_tc_tiling_on_sc=False),
        scratch_types=_SPMM_SCRATCH,
    )


def _xw_body(x_ref, w_ref, o_ref):
    o_ref[...] = jnp.dot(x_ref[...], w_ref[...],
                         preferred_element_type=jnp.float32)


def _u_body(p_ref, w23_ref, o_ref):
    h = jnp.maximum(p_ref[0] + p_ref[1], 0.0)
    o_ref[...] = jnp.dot(h, w23_ref[...], preferred_element_type=jnp.float32)


def _z_body(q_ref, noise_ref, o_ref):
    m = q_ref[0] + q_ref[1]
    o_ref[...] = m[:, :N_HID2] + jnp.exp(m[:, N_HID2:]) * noise_ref[...]


BM = 200


def _decode_body(zi_ref, zj_ref, o_ref):
    s = lax.dot_general(zi_ref[...], zj_ref[...], (((1,), (1,)), ((), ())),
                        preferred_element_type=jnp.float32)
    o_ref[...] = jax.nn.sigmoid(s)


def kernel(x, edge_index, edge_weight, noise, W1, W2, W3):
    pad = (0, E_PAD - N_EDGES)
    src = jnp.pad(edge_index[0].astype(jnp.int32), pad).reshape(EROWS, CHUNK)
    dst = jnp.pad(edge_index[1].astype(jnp.int32), pad).reshape(EROWS, CHUNK)
    w = jnp.pad(edge_weight.astype(jnp.float32), pad).reshape(EROWS, CHUNK)
    w23 = jnp.concatenate([W2, W3], axis=1)

    spmm = _get_spmm_sc()

    xw = pl.pallas_call(
        _xw_body,
        out_shape=jax.ShapeDtypeStruct((N_NODES, N_HID1), jnp.float32),
    )(x, W1)

    p = spmm(xw, src, dst, w)

    u = pl.pallas_call(
        _u_body,
        out_shape=jax.ShapeDtypeStruct((N_NODES, N_HID1), jnp.float32),
    )(p, w23)

    q = spmm(u, src, dst, w)

    z = pl.pallas_call(
        _z_body,
        out_shape=jax.ShapeDtypeStruct((N_NODES, N_HID2), jnp.float32),
    )(q, noise)

    a_pred = pl.pallas_call(
        _decode_body,
        grid=(N_NODES // BM,),
        in_specs=[
            pl.BlockSpec((BM, N_HID2), lambda i: (i, 0)),
            pl.BlockSpec((N_NODES, N_HID2), lambda i: (0, 0)),
        ],
        out_specs=pl.BlockSpec((BM, N_NODES), lambda i: (i, 0)),
        out_shape=jax.ShapeDtypeStruct((N_NODES, N_NODES), jnp.float32),
    )(z, z)

    return a_pred


# speedup vs baseline: 9.4468x; 1.2738x over previous
import functools

import jax
import jax.numpy as jnp
from jax import lax
from jax.experimental import pallas as pl
from jax.experimental.pallas import tpu as pltpu
from jax.experimental.pallas import tpu_sc as plsc

N_NODES = 10000
N_EDGES = 160000
N_FEATS = 128
N_HID1 = 32
N_HID2 = 16

NC = 2
NS = 16
NW = NC * NS
CHUNK = 128
N_CHUNKS = N_EDGES // CHUNK
RING_CHUNKS = 39
MAX_CHUNKS = 40
EROWS = 1256
NBUF = 3
ROWS_A = 624
ROWS_B = N_NODES - 15 * ROWS_A


_SPMM_SCRATCH = (
    [pltpu.VMEM_SHARED((N_NODES, N_HID1), jnp.float32)]
    + [pltpu.VMEM((MAX_CHUNKS, CHUNK), jnp.int32)] * 2
    + [pltpu.VMEM((MAX_CHUNKS, CHUNK), jnp.float32)]
    + [pltpu.VMEM((CHUNK, N_HID1), jnp.float32)] * NBUF
    + [pltpu.SemaphoreType.DMA] * (2 * NBUF)
    + [pltpu.VMEM((ROWS_B, N_HID1), jnp.float32)]
)


def _spmm_body(m_hbm, src_hbm, dst_hbm, w_hbm, out_hbm,
               acc_shared, src_all, dst_all, w_all,
               rows0, rows1, rows2,
               g0, g1, g2, s0, s1, s2, buf_v):
    rows = [rows0, rows1, rows2]
    gsem = [g0, g1, g2]
    ssem = [s0, s1, s2]
    cid = lax.axis_index("c")
    sid = lax.axis_index("s")
    wid = sid * NC + cid
    row0 = jnp.where(wid < 2, 40 * wid, 80 + RING_CHUNKS * (wid - 2))

    pltpu.sync_copy(src_hbm.at[pl.ds(row0, MAX_CHUNKS), :], src_all)
    pltpu.sync_copy(dst_hbm.at[pl.ds(row0, MAX_CHUNKS), :], dst_all)
    pltpu.sync_copy(w_hbm.at[pl.ds(row0, MAX_CHUNKS), :], w_all)

    zero16 = jnp.zeros((16,), jnp.float32)

    def zero_row(r, carry):
        buf_v[r, 0:16] = zero16
        buf_v[r, 16:32] = zero16
        return carry

    lax.fori_loop(0, ROWS_B, zero_row, 0)

    @pl.when(sid < 15)
    def _():
        pltpu.sync_copy(buf_v.at[pl.ds(0, ROWS_A), :],
                        acc_shared.at[pl.ds(sid * ROWS_A, ROWS_A), :])

    @pl.when(sid == 15)
    def _():
        pltpu.sync_copy(buf_v, acc_shared.at[pl.ds(15 * ROWS_A, ROWS_B), :])

    plsc.subcore_barrier()

    def gather_start(c, b):
        pltpu.async_copy(m_hbm.at[src_all.at[c]], rows[b], gsem[b])

    def wait_chunk(sem, b):
        pltpu.make_async_copy(m_hbm.at[pl.ds(0, CHUNK), :], rows[b], sem).wait()

    def scale_rows(c, b):
        def scale_group(j, carry):
            w16 = w_all[c, pl.ds(j * 16, 16)]
            for i in range(16):
                e = j * 16 + i
                w = w16[i]
                rows[b][e, 0:16] = rows[b][e, 0:16] * w
                rows[b][e, 16:32] = rows[b][e, 16:32] * w
            return carry

        lax.fori_loop(0, CHUNK // 16, scale_group, 0)

    for b in range(NBUF - 1):
        gather_start(b, b)

    def outer(kk, carry):
        for b in range(NBUF):
            c = kk * NBUF + b
            bp = (b + NBUF - 1) % NBUF

            @pl.when(c + NBUF - 1 < RING_CHUNKS)
            def _():
                @pl.when(c >= 1)
                def _():
                    wait_chunk(ssem[bp], bp)

                gather_start(c + NBUF - 1, bp)

            wait_chunk(gsem[b], b)
            scale_rows(c, b)
            pltpu.async_copy(rows[b], acc_shared.at[dst_all.at[c]],
                             ssem[b], add=True)
        return carry

    lax.fori_loop(0, RING_CHUNKS // NBUF, outer, 0)

    for b in range(NBUF):
        wait_chunk(ssem[b], b)

    @pl.when(wid < 2)
    def _():
        gather_start(RING_CHUNKS, 0)
        wait_chunk(gsem[0], 0)
        scale_rows(RING_CHUNKS, 0)
        pltpu.async_copy(rows[0], acc_shared.at[dst_all.at[RING_CHUNKS]],
                         ssem[0], add=True)
        wait_chunk(ssem[0], 0)

    plsc.subcore_barrier()

    @pl.when(sid < 15)
    def _():
        pltpu.sync_copy(acc_shared.at[pl.ds(sid * ROWS_A, ROWS_A), :],
                        buf_v.at[pl.ds(0, ROWS_A), :])
        pltpu.sync_copy(buf_v.at[pl.ds(0, ROWS_A), :],
                        out_hbm.at[cid, pl.ds(sid * ROWS_A, ROWS_A), :])

    @pl.when(sid == 15)
    def _():
        pltpu.sync_copy(acc_shared.at[pl.ds(15 * ROWS_A, ROWS_B), :], buf_v)
        pltpu.sync_copy(buf_v, out_hbm.at[cid, pl.ds(15 * ROWS_A, ROWS_B), :])


@functools.lru_cache(maxsize=None)
def _get_spmm_sc():
    return pl.kernel(
        _spmm_body,
        out_type=jax.ShapeDtypeStruct((NC, N_NODES, N_HID1), jnp.float32),
        mesh=plsc.VectorSubcoreMesh(
            core_axis_name="c", subcore_axis_name="s",
            num_cores=NC, num_subcores=NS),
        compiler_params=pltpu.CompilerParams(use_tc_tiling_on_sc=False),
        scratch_types=_SPMM_SCRATCH,
    )


def _xw_body(x_ref, w_ref, o_ref):
    o_ref[...] = jnp.dot(x_ref[...], w_ref[...],
                         preferred_element_type=jnp.float32)


def _u_body(p_ref, w23_ref, o_ref):
    h = jnp.maximum(p_ref[0] + p_ref[1], 0.0)
    o_ref[...] = jnp.dot(h, w23_ref[...], preferred_element_type=jnp.float32)


def _z_body(q_ref, noise_ref, o_ref):
    m = q_ref[0] + q_ref[1]
    o_ref[...] = m[:, :N_HID2] + jnp.exp(m[:, N_HID2:]) * noise_ref[...]


BM = 200


def _decode_body(zi_ref, zj_ref, o_ref):
    s = lax.dot_general(zi_ref[...], zj_ref[...], (((1,), (1,)), ((), ())),
                        preferred_element_type=jnp.float32)
    o_ref[...] = jax.nn.sigmoid(s)


def kernel(x, edge_index, edge_weight, noise, W1, W2, W3):
    pad = (0, EROWS * CHUNK - N_EDGES)
    src = jnp.pad(edge_index[0].astype(jnp.int32), pad).reshape(EROWS, CHUNK)
    dst = jnp.pad(edge_index[1].astype(jnp.int32), pad).reshape(EROWS, CHUNK)
    w = jnp.pad(edge_weight.astype(jnp.float32), pad).reshape(EROWS, CHUNK)
    w23 = jnp.concatenate([W2, W3], axis=1)

    spmm = _get_spmm_sc()

    xw = pl.pallas_call(
        _xw_body,
        out_shape=jax.ShapeDtypeStruct((N_NODES, N_HID1), jnp.float32),
    )(x, W1)

    p = spmm(xw, src, dst, w)

    u = pl.pallas_call(
        _u_body,
        out_shape=jax.ShapeDtypeStruct((N_NODES, N_HID1), jnp.float32),
    )(p, w23)

    q = spmm(u, src, dst, w)

    z = pl.pallas_call(
        _z_body,
        out_shape=jax.ShapeDtypeStruct((N_NODES, N_HID2), jnp.float32),
    )(q, noise)

    a_pred = pl.pallas_call(
        _decode_body,
        grid=(N_NODES // BM,),
        in_specs=[
            pl.BlockSpec((BM, N_HID2), lambda i: (i, 0)),
            pl.BlockSpec((N_NODES, N_HID2), lambda i: (0, 0)),
        ],
        out_specs=pl.BlockSpec((BM, N_NODES), lambda i: (i, 0)),
        out_shape=jax.ShapeDtypeStruct((N_NODES, N_NODES), jnp.float32),
    )(z, z)

    return a_pred


# speedup vs baseline: 9.7332x; 1.0303x over previous
import functools

import jax
import jax.numpy as jnp
from jax import lax
from jax.experimental import pallas as pl
from jax.experimental.pallas import tpu as pltpu
from jax.experimental.pallas import tpu_sc as plsc

N_NODES = 10000
N_EDGES = 160000
N_FEATS = 128
N_HID1 = 32
N_HID2 = 16

NC = 2
NS = 16
NW = NC * NS
CHUNK = 128
N_CHUNKS = N_EDGES // CHUNK
RING_CHUNKS = 39
MAX_CHUNKS = 40
EROWS = 1256
NBUF = 3
ROWS_A = 624
ROWS_B = N_NODES - 15 * ROWS_A


_SPMM_SCRATCH = (
    [pltpu.VMEM_SHARED((N_NODES, N_HID1), jnp.float32)]
    + [pltpu.VMEM((MAX_CHUNKS, CHUNK), jnp.int32)] * 2
    + [pltpu.VMEM((MAX_CHUNKS, CHUNK), jnp.float32)]
    + [pltpu.VMEM((CHUNK, N_HID1), jnp.float32)] * NBUF
    + [pltpu.SemaphoreType.DMA] * (2 * NBUF)
    + [pltpu.VMEM((ROWS_B, N_HID1), jnp.float32)]
)


def _spmm_body(m_hbm, src_hbm, dst_hbm, w_hbm, out_hbm,
               acc_shared, src_all, dst_all, w_all,
               rows0, rows1, rows2,
               g0, g1, g2, s0, s1, s2, buf_v):
    rows = [rows0, rows1, rows2]
    gsem = [g0, g1, g2]
    ssem = [s0, s1, s2]
    cid = lax.axis_index("c")
    sid = lax.axis_index("s")
    wid = sid * NC + cid
    row0 = jnp.where(wid < 2, 40 * wid, 80 + RING_CHUNKS * (wid - 2))

    pltpu.sync_copy(src_hbm.at[pl.ds(row0, MAX_CHUNKS), :], src_all)
    pltpu.sync_copy(dst_hbm.at[pl.ds(row0, MAX_CHUNKS), :], dst_all)
    pltpu.sync_copy(w_hbm.at[pl.ds(row0, MAX_CHUNKS), :], w_all)

    zero16 = jnp.zeros((16,), jnp.float32)

    def zero_row(r, carry):
        buf_v[r, 0:16] = zero16
        buf_v[r, 16:32] = zero16
        return carry

    lax.fori_loop(0, ROWS_B, zero_row, 0)

    @pl.when(sid < 15)
    def _():
        pltpu.sync_copy(buf_v.at[pl.ds(0, ROWS_A), :],
                        acc_shared.at[pl.ds(sid * ROWS_A, ROWS_A), :])

    @pl.when(sid == 15)
    def _():
        pltpu.sync_copy(buf_v, acc_shared.at[pl.ds(15 * ROWS_A, ROWS_B), :])

    plsc.subcore_barrier()

    def gather_start(c, b):
        pltpu.async_copy(m_hbm.at[src_all.at[c]], rows[b], gsem[b])

    def wait_chunk(sem, b):
        pltpu.make_async_copy(m_hbm.at[pl.ds(0, CHUNK), :], rows[b], sem).wait()

    def scale_rows(c, b):
        def scale_group(j, carry):
            w16 = w_all[c, pl.ds(j * 16, 16)]
            for i in range(16):
                e = j * 16 + i
                w = w16[i]
                rows[b][e, 0:16] = rows[b][e, 0:16] * w
                rows[b][e, 16:32] = rows[b][e, 16:32] * w
            return carry

        lax.fori_loop(0, CHUNK // 16, scale_group, 0)

    for b in range(NBUF - 1):
        gather_start(b, b)

    def outer(kk, carry):
        for b in range(NBUF):
            c = kk * NBUF + b
            bp = (b + NBUF - 1) % NBUF

            @pl.when(c + NBUF - 1 < RING_CHUNKS)
            def _():
                @pl.when(c >= 1)
                def _():
                    wait_chunk(ssem[bp], bp)

                gather_start(c + NBUF - 1, bp)

            wait_chunk(gsem[b], b)
            scale_rows(c, b)
            pltpu.async_copy(rows[b], acc_shared.at[dst_all.at[c]],
                             ssem[b], add=True)
        return carry

    lax.fori_loop(0, RING_CHUNKS // NBUF, outer, 0)

    for b in range(NBUF):
        wait_chunk(ssem[b], b)

    @pl.when(wid < 2)
    def _():
        gather_start(RING_CHUNKS, 0)
        wait_chunk(gsem[0], 0)
        scale_rows(RING_CHUNKS, 0)
        pltpu.async_copy(rows[0], acc_shared.at[dst_all.at[RING_CHUNKS]],
                         ssem[0], add=True)
        wait_chunk(ssem[0], 0)

    plsc.subcore_barrier()

    @pl.when(sid < 15)
    def _():
        pltpu.sync_copy(acc_shared.at[pl.ds(sid * ROWS_A, ROWS_A), :],
                        buf_v.at[pl.ds(0, ROWS_A), :])
        pltpu.sync_copy(buf_v.at[pl.ds(0, ROWS_A), :],
                        out_hbm.at[cid, pl.ds(sid * ROWS_A, ROWS_A), :])

    @pl.when(sid == 15)
    def _():
        pltpu.sync_copy(acc_shared.at[pl.ds(15 * ROWS_A, ROWS_B), :], buf_v)
        pltpu.sync_copy(buf_v, out_hbm.at[cid, pl.ds(15 * ROWS_A, ROWS_B), :])


@functools.lru_cache(maxsize=None)
def _get_spmm_sc():
    return pl.kernel(
        _spmm_body,
        out_type=jax.ShapeDtypeStruct((NC, N_NODES, N_HID1), jnp.float32),
        mesh=plsc.VectorSubcoreMesh(
            core_axis_name="c", subcore_axis_name="s",
            num_cores=NC, num_subcores=NS),
        compiler_params=pltpu.CompilerParams(use_tc_tiling_on_sc=False),
        scratch_types=_SPMM_SCRATCH,
    )


def _xw_body(x_ref, w_ref, o_ref):
    o_ref[...] = jnp.dot(x_ref[...], w_ref[...],
                         preferred_element_type=jnp.float32)


def _u_body(p_ref, w23_ref, o_ref):
    h = jnp.maximum(p_ref[0] + p_ref[1], 0.0)
    o_ref[...] = jnp.dot(h, w23_ref[...], preferred_element_type=jnp.float32)


def _z_body(q_ref, noise_ref, o_ref):
    m = q_ref[0] + q_ref[1]
    o_ref[...] = m[:, :N_HID2] + jnp.exp(m[:, N_HID2:]) * noise_ref[...]


BM = 400


def _decode_body(zi_ref, zj_ref, o_ref):
    s = lax.dot_general(zi_ref[...], zj_ref[...], (((1,), (1,)), ((), ())),
                        preferred_element_type=jnp.float32)
    o_ref[...] = jax.nn.sigmoid(s)


def kernel(x, edge_index, edge_weight, noise, W1, W2, W3):
    pad = (0, EROWS * CHUNK - N_EDGES)
    src = jnp.pad(edge_index[0].astype(jnp.int32), pad).reshape(EROWS, CHUNK)
    dst = jnp.pad(edge_index[1].astype(jnp.int32), pad).reshape(EROWS, CHUNK)
    w = jnp.pad(edge_weight.astype(jnp.float32), pad).reshape(EROWS, CHUNK)
    w23 = jnp.concatenate([W2, W3], axis=1)

    spmm = _get_spmm_sc()

    xw = pl.pallas_call(
        _xw_body,
        out_shape=jax.ShapeDtypeStruct((N_NODES, N_HID1), jnp.float32),
    )(x, W1)

    p = spmm(xw, src, dst, w)

    u = pl.pallas_call(
        _u_body,
        out_shape=jax.ShapeDtypeStruct((N_NODES, N_HID1), jnp.float32),
    )(p, w23)

    q = spmm(u, src, dst, w)

    z = pl.pallas_call(
        _z_body,
        out_shape=jax.ShapeDtypeStruct((N_NODES, N_HID2), jnp.float32),
    )(q, noise)

    a_pred = pl.pallas_call(
        _decode_body,
        grid=(N_NODES // BM,),
        in_specs=[
            pl.BlockSpec((BM, N_HID2), lambda i: (i, 0)),
            pl.BlockSpec((N_NODES, N_HID2), lambda i: (0, 0)),
        ],
        out_specs=pl.BlockSpec((BM, N_NODES), lambda i: (i, 0)),
        out_shape=jax.ShapeDtypeStruct((N_NODES, N_NODES), jnp.float32),
    )(z, z)

    return a_pred


# speedup vs baseline: 9.9415x; 1.0214x over previous
import functools

import jax
import jax.numpy as jnp
from jax import lax
from jax.experimental import pallas as pl
from jax.experimental.pallas import tpu as pltpu
from jax.experimental.pallas import tpu_sc as plsc

N_NODES = 10000
N_EDGES = 160000
N_FEATS = 128
N_HID1 = 32
N_HID2 = 16

NC = 2
NS = 16
NW = NC * NS
CHUNK = 128
N_CHUNKS = N_EDGES // CHUNK
RING_CHUNKS = 39
NBUF = 3
ROWS_A = 624
ROWS_B = N_NODES - 15 * ROWS_A


_SPMM_SCRATCH = (
    [pltpu.VMEM_SHARED((N_NODES, N_HID1), jnp.float32)]
    + [pltpu.VMEM((RING_CHUNKS * CHUNK,), jnp.int32)]
    + [pltpu.VMEM((RING_CHUNKS, CHUNK), jnp.int32)]
    + [pltpu.VMEM((RING_CHUNKS * CHUNK,), jnp.float32)]
    + [pltpu.VMEM((CHUNK,), jnp.int32)]
    + [pltpu.VMEM((1, CHUNK), jnp.int32)]
    + [pltpu.VMEM((CHUNK,), jnp.float32)]
    + [pltpu.VMEM((CHUNK, N_HID1), jnp.float32)] * NBUF
    + [pltpu.SemaphoreType.DMA] * (2 * NBUF)
    + [pltpu.VMEM((ROWS_B, N_HID1), jnp.float32)]
)


def _spmm_body(m_hbm, src_hbm, dst_hbm, w_hbm, out_hbm,
               acc_shared, src_all, dst_all, w_all, src_x, dst_x, w_x,
               rows0, rows1, rows2,
               g0, g1, g2, s0, s1, s2, buf_v):
    rows = [rows0, rows1, rows2]
    gsem = [g0, g1, g2]
    ssem = [s0, s1, s2]
    cid = lax.axis_index("c")
    sid = lax.axis_index("s")
    wid = sid * NC + cid
    row0 = RING_CHUNKS * wid

    pltpu.sync_copy(src_hbm.at[pl.ds(row0 * CHUNK, RING_CHUNKS * CHUNK)], src_all)
    pltpu.sync_copy(dst_hbm.at[pl.ds(row0, RING_CHUNKS), :], dst_all)
    pltpu.sync_copy(w_hbm.at[pl.ds(row0 * CHUNK, RING_CHUNKS * CHUNK)], w_all)

    @pl.when(wid < 2)
    def _():
        ex = RING_CHUNKS * NW + wid
        pltpu.sync_copy(src_hbm.at[pl.ds(ex * CHUNK, CHUNK)], src_x)
        pltpu.sync_copy(dst_hbm.at[pl.ds(ex, 1), :], dst_x)
        pltpu.sync_copy(w_hbm.at[pl.ds(ex * CHUNK, CHUNK)], w_x)

    zero16 = jnp.zeros((16,), jnp.float32)

    def zero_row(r, carry):
        buf_v[r, 0:16] = zero16
        buf_v[r, 16:32] = zero16
        return carry

    lax.fori_loop(0, ROWS_B, zero_row, 0)

    @pl.when(sid < 15)
    def _():
        pltpu.sync_copy(buf_v.at[pl.ds(0, ROWS_A), :],
                        acc_shared.at[pl.ds(sid * ROWS_A, ROWS_A), :])

    @pl.when(sid == 15)
    def _():
        pltpu.sync_copy(buf_v, acc_shared.at[pl.ds(15 * ROWS_A, ROWS_B), :])

    plsc.subcore_barrier()

    def gather_start(c, b):
        pltpu.async_copy(m_hbm.at[src_all.at[pl.ds(c * CHUNK, CHUNK)]],
                         rows[b], gsem[b])

    def wait_chunk(sem, b):
        pltpu.make_async_copy(m_hbm.at[pl.ds(0, CHUNK), :], rows[b], sem).wait()

    def scale_rows(wref, base, b):
        def scale_group(j, carry):
            w16 = wref[pl.ds(base + j * 16, 16)]
            for i in range(16):
                e = j * 16 + i
                w = w16[i]
                rows[b][e, 0:16] = rows[b][e, 0:16] * w
                rows[b][e, 16:32] = rows[b][e, 16:32] * w
            return carry

        lax.fori_loop(0, CHUNK // 16, scale_group, 0)

    for b in range(NBUF - 1):
        gather_start(b, b)

    def outer(kk, carry):
        for b in range(NBUF):
            c = kk * NBUF + b
            bp = (b + NBUF - 1) % NBUF

            @pl.when(c + NBUF - 1 < RING_CHUNKS)
            def _():
                @pl.when(c >= 1)
                def _():
                    wait_chunk(ssem[bp], bp)

                gather_start(c + NBUF - 1, bp)

            wait_chunk(gsem[b], b)
            scale_rows(w_all, c * CHUNK, b)
            pltpu.async_copy(rows[b], acc_shared.at[dst_all.at[c]],
                             ssem[b], add=True)
        return carry

    lax.fori_loop(0, RING_CHUNKS // NBUF, outer, 0)

    for b in range(NBUF):
        wait_chunk(ssem[b], b)

    @pl.when(wid < 2)
    def _():
        pltpu.async_copy(m_hbm.at[src_x], rows[0], gsem[0])
        wait_chunk(gsem[0], 0)
        scale_rows(w_x, 0, 0)
        pltpu.async_copy(rows[0], acc_shared.at[dst_x.at[0]],
                         ssem[0], add=True)
        wait_chunk(ssem[0], 0)

    plsc.subcore_barrier()

    @pl.when(sid < 15)
    def _():
        pltpu.sync_copy(acc_shared.at[pl.ds(sid * ROWS_A, ROWS_A), :],
                        buf_v.at[pl.ds(0, ROWS_A), :])
        pltpu.sync_copy(buf_v.at[pl.ds(0, ROWS_A), :],
                        out_hbm.at[cid, pl.ds(sid * ROWS_A, ROWS_A), :])

    @pl.when(sid == 15)
    def _():
        pltpu.sync_copy(acc_shared.at[pl.ds(15 * ROWS_A, ROWS_B), :], buf_v)
        pltpu.sync_copy(buf_v, out_hbm.at[cid, pl.ds(15 * ROWS_A, ROWS_B), :])


@functools.lru_cache(maxsize=None)
def _get_spmm_sc():
    return pl.kernel(
        _spmm_body,
        out_type=jax.ShapeDtypeStruct((NC, N_NODES, N_HID1), jnp.float32),
        mesh=plsc.VectorSubcoreMesh(
            core_axis_name="c", subcore_axis_name="s",
            num_cores=NC, num_subcores=NS),
        compiler_params=pltpu.CompilerParams(use_tc_tiling_on_sc=False),
        scratch_types=_SPMM_SCRATCH,
    )


def _xw_body(x_ref, w_ref, o_ref):
    o_ref[...] = jnp.dot(x_ref[...], w_ref[...],
                         preferred_element_type=jnp.float32)


def _u_body(p_ref, w23_ref, o_ref):
    h = jnp.maximum(p_ref[0] + p_ref[1], 0.0)
    o_ref[...] = jnp.dot(h, w23_ref[...], preferred_element_type=jnp.float32)


BM = 400


def _decode_body(q_ref, noise_ref, o_ref, z_s):
    @pl.when(pl.program_id(0) == 0)
    def _():
        m = q_ref[0] + q_ref[1]
        z_s[...] = m[:, :N_HID2] + jnp.exp(m[:, N_HID2:]) * noise_ref[...]

    zi = z_s[pl.ds(pl.program_id(0) * BM, BM), :]
    s = lax.dot_general(zi, z_s[...], (((1,), (1,)), ((), ())),
                        preferred_element_type=jnp.float32)
    o_ref[...] = jax.nn.sigmoid(s)


def kernel(x, edge_index, edge_weight, noise, W1, W2, W3):
    src = edge_index[0].astype(jnp.int32)
    dst = edge_index[1].astype(jnp.int32).reshape(N_CHUNKS, CHUNK)
    w = edge_weight.astype(jnp.float32)
    w23 = jnp.concatenate([W2, W3], axis=1)

    spmm = _get_spmm_sc()

    xw = pl.pallas_call(
        _xw_body,
        out_shape=jax.ShapeDtypeStruct((N_NODES, N_HID1), jnp.float32),
    )(x, W1)

    p = spmm(xw, src, dst, w)

    u = pl.pallas_call(
        _u_body,
        out_shape=jax.ShapeDtypeStruct((N_NODES, N_HID1), jnp.float32),
    )(p, w23)

    q = spmm(u, src, dst, w)

    a_pred = pl.pallas_call(
        _decode_body,
        grid=(N_NODES // BM,),
        in_specs=[
            pl.BlockSpec((NC, N_NODES, N_HID1), lambda i: (0, 0, 0)),
            pl.BlockSpec((N_NODES, N_HID2), lambda i: (0, 0)),
        ],
        out_specs=pl.BlockSpec((BM, N_NODES), lambda i: (i, 0)),
        out_shape=jax.ShapeDtypeStruct((N_NODES, N_NODES), jnp.float32),
        scratch_shapes=[pltpu.VMEM((N_NODES, N_HID2), jnp.float32)],
    )(q, noise)

    return a_pred


# speedup vs baseline: 10.4104x; 1.0472x over previous
import functools

import jax
import jax.numpy as jnp
from jax import lax
from jax.experimental import pallas as pl
from jax.experimental.pallas import tpu as pltpu
from jax.experimental.pallas import tpu_sc as plsc

N_NODES = 10000
N_EDGES = 160000
N_FEATS = 128
N_HID1 = 32
N_HID2 = 16

NC = 2
NS = 16
NW = NC * NS
CHUNK = 128
N_CHUNKS = N_EDGES // CHUNK
RING_CHUNKS = 39
NBUF = 3
ROWS_A = 624
ROWS_B = N_NODES - 15 * ROWS_A


_SPMM_SCRATCH = (
    [pltpu.VMEM_SHARED((N_NODES, N_HID1), jnp.float32)]
    + [pltpu.VMEM((RING_CHUNKS * CHUNK,), jnp.int32)]
    + [pltpu.VMEM((RING_CHUNKS, CHUNK), jnp.int32)]
    + [pltpu.VMEM((RING_CHUNKS * CHUNK,), jnp.float32)]
    + [pltpu.VMEM((CHUNK,), jnp.int32)]
    + [pltpu.VMEM((1, CHUNK), jnp.int32)]
    + [pltpu.VMEM((CHUNK,), jnp.float32)]
    + [pltpu.VMEM((CHUNK, N_HID1), jnp.float32)] * NBUF
    + [pltpu.SemaphoreType.DMA] * (2 * NBUF)
    + [pltpu.VMEM((ROWS_B, N_HID1), jnp.float32)]
)


def _spmm_body(m_hbm, src_hbm, dst_hbm, w_hbm, out_hbm,
               acc_shared, src_all, dst_all, w_all, src_x, dst_x, w_x,
               rows0, rows1, rows2,
               g0, g1, g2, s0, s1, s2, buf_v):
    rows = [rows0, rows1, rows2]
    gsem = [g0, g1, g2]
    ssem = [s0, s1, s2]
    cid = lax.axis_index("c")
    sid = lax.axis_index("s")
    wid = sid * NC + cid
    row0 = RING_CHUNKS * wid

    pltpu.sync_copy(src_hbm.at[pl.ds(row0 * CHUNK, RING_CHUNKS * CHUNK)], src_all)
    pltpu.sync_copy(dst_hbm.at[pl.ds(row0, RING_CHUNKS), :], dst_all)
    pltpu.sync_copy(w_hbm.at[pl.ds(row0 * CHUNK, RING_CHUNKS * CHUNK)], w_all)

    @pl.when(wid < 2)
    def _():
        ex = RING_CHUNKS * NW + wid
        pltpu.sync_copy(src_hbm.at[pl.ds(ex * CHUNK, CHUNK)], src_x)
        pltpu.sync_copy(dst_hbm.at[pl.ds(ex, 1), :], dst_x)
        pltpu.sync_copy(w_hbm.at[pl.ds(ex * CHUNK, CHUNK)], w_x)

    zero16 = jnp.zeros((16,), jnp.float32)

    def zero_row(r, carry):
        buf_v[r, 0:16] = zero16
        buf_v[r, 16:32] = zero16
        return carry

    lax.fori_loop(0, ROWS_B, zero_row, 0)

    @pl.when(sid < 15)
    def _():
        pltpu.sync_copy(buf_v.at[pl.ds(0, ROWS_A), :],
                        acc_shared.at[pl.ds(sid * ROWS_A, ROWS_A), :])

    @pl.when(sid == 15)
    def _():
        pltpu.sync_copy(buf_v, acc_shared.at[pl.ds(15 * ROWS_A, ROWS_B), :])

    plsc.subcore_barrier()

    def gather_start(c, b):
        pltpu.async_copy(m_hbm.at[src_all.at[pl.ds(c * CHUNK, CHUNK)]],
                         rows[b], gsem[b])

    def wait_chunk(sem, b):
        pltpu.make_async_copy(m_hbm.at[pl.ds(0, CHUNK), :], rows[b], sem).wait()

    def scale_rows(wref, base, b):
        def scale_group(j, carry):
            w16 = wref[pl.ds(base + j * 16, 16)]
            for i in range(16):
                e = j * 16 + i
                w = w16[i]
                rows[b][e, 0:16] = rows[b][e, 0:16] * w
                rows[b][e, 16:32] = rows[b][e, 16:32] * w
            return carry

        lax.fori_loop(0, CHUNK // 16, scale_group, 0)

    for b in range(NBUF - 1):
        gather_start(b, b)

    def outer(kk, carry):
        for b in range(NBUF):
            c = kk * NBUF + b
            bp = (b + NBUF - 1) % NBUF

            @pl.when(c + NBUF - 1 < RING_CHUNKS)
            def _():
                @pl.when(c >= 1)
                def _():
                    wait_chunk(ssem[bp], bp)

                gather_start(c + NBUF - 1, bp)

            wait_chunk(gsem[b], b)
            scale_rows(w_all, c * CHUNK, b)
            pltpu.async_copy(rows[b], acc_shared.at[dst_all.at[c]],
                             ssem[b], add=True)
        return carry

    lax.fori_loop(0, RING_CHUNKS // NBUF, outer, 0)

    for b in range(NBUF):
        wait_chunk(ssem[b], b)

    @pl.when(wid < 2)
    def _():
        pltpu.async_copy(m_hbm.at[src_x], rows[0], gsem[0])
        wait_chunk(gsem[0], 0)
        scale_rows(w_x, 0, 0)
        pltpu.async_copy(rows[0], acc_shared.at[dst_x.at[0]],
                         ssem[0], add=True)
        wait_chunk(ssem[0], 0)

    plsc.subcore_barrier()

    @pl.when(sid < 15)
    def _():
        pltpu.sync_copy(acc_shared.at[pl.ds(sid * ROWS_A, ROWS_A), :],
                        buf_v.at[pl.ds(0, ROWS_A), :])
        pltpu.sync_copy(buf_v.at[pl.ds(0, ROWS_A), :],
                        out_hbm.at[cid, pl.ds(sid * ROWS_A, ROWS_A), :])

    @pl.when(sid == 15)
    def _():
        pltpu.sync_copy(acc_shared.at[pl.ds(15 * ROWS_A, ROWS_B), :], buf_v)
        pltpu.sync_copy(buf_v, out_hbm.at[cid, pl.ds(15 * ROWS_A, ROWS_B), :])


@functools.lru_cache(maxsize=None)
def _get_spmm_sc():
    return pl.kernel(
        _spmm_body,
        out_type=jax.ShapeDtypeStruct((NC, N_NODES, N_HID1), jnp.float32),
        mesh=plsc.VectorSubcoreMesh(
            core_axis_name="c", subcore_axis_name="s",
            num_cores=NC, num_subcores=NS),
        compiler_params=pltpu.CompilerParams(use_tc_tiling_on_sc=False),
        scratch_types=_SPMM_SCRATCH,
    )


def _xw_body(x_ref, w_ref, o_ref):
    o_ref[...] = jnp.dot(x_ref[...], w_ref[...],
                         preferred_element_type=jnp.float32)


def _u_body(p_ref, w23_ref, o_ref):
    h = jnp.maximum(p_ref[0] + p_ref[1], 0.0)
    o_ref[...] = jnp.dot(h, w23_ref[...], preferred_element_type=jnp.float32)


BM = 400


def _decode_body(q_ref, noise_ref, o_ref, z_s):
    @pl.when(pl.program_id(0) == 0)
    def _():
        m = q_ref[0] + q_ref[1]
        z_s[...] = m[:, :N_HID2] + jnp.exp(m[:, N_HID2:]) * noise_ref[...]

    zi = z_s[pl.ds(pl.program_id(0) * BM, BM), :]
    s = lax.dot_general(zi, z_s[...], (((1,), (1,)), ((), ())),
                        preferred_element_type=jnp.float32)
    o_ref[...] = 0.5 * jnp.tanh(0.5 * s) + 0.5


def kernel(x, edge_index, edge_weight, noise, W1, W2, W3):
    src = edge_index[0].astype(jnp.int32)
    dst = edge_index[1].astype(jnp.int32).reshape(N_CHUNKS, CHUNK)
    w = edge_weight.astype(jnp.float32)
    w23 = jnp.concatenate([W2, W3], axis=1)

    spmm = _get_spmm_sc()

    xw = pl.pallas_call(
        _xw_body,
        out_shape=jax.ShapeDtypeStruct((N_NODES, N_HID1), jnp.float32),
    )(x, W1)

    p = spmm(xw, src, dst, w)

    u = pl.pallas_call(
        _u_body,
        out_shape=jax.ShapeDtypeStruct((N_NODES, N_HID1), jnp.float32),
    )(p, w23)

    q = spmm(u, src, dst, w)

    a_pred = pl.pallas_call(
        _decode_body,
        grid=(N_NODES // BM,),
        in_specs=[
            pl.BlockSpec((NC, N_NODES, N_HID1), lambda i: (0, 0, 0)),
            pl.BlockSpec((N_NODES, N_HID2), lambda i: (0, 0)),
        ],
        out_specs=pl.BlockSpec((BM, N_NODES), lambda i: (i, 0)),
        out_shape=jax.ShapeDtypeStruct((N_NODES, N_NODES), jnp.float32),
        scratch_shapes=[pltpu.VMEM((N_NODES, N_HID2), jnp.float32)],
    )(q, noise)

    return a_pred
